# baseline (speedup 1.0000x reference)
import jax
import jax.numpy as jnp
from jax.experimental import pallas as pl


def _body(x_ref, kv_ref, m_ref, o_ref):
    wt = kv_ref[...] * m_ref[...]
    o_ref[...] = jnp.dot(wt, x_ref[...], preferred_element_type=jnp.float32)


def kernel(inputs, kernel_values, mask):
    b, c, h, w = inputs.shape
    f = kernel_values.shape[0]
    hw = h * w
    flat_inputs = jnp.reshape(inputs, (c, b * hw))
    out2d = pl.pallas_call(
        _body,
        grid=(b,),
        in_specs=[
            pl.BlockSpec((c, hw), lambda i: (0, i)),
            pl.BlockSpec((f, c), lambda i: (0, 0)),
            pl.BlockSpec((f, c), lambda i: (0, 0)),
        ],
        out_specs=pl.BlockSpec((f, hw), lambda i: (i, 0)),
        out_shape=jax.ShapeDtypeStruct((b * f, hw), jnp.float32),
    )(flat_inputs, kernel_values, mask)
    parts = [
        jnp.reshape(out2d[i * f:(i + 1) * f], (f, h, w)) for i in range(b)
    ]
    return jnp.stack(parts, axis=0)


# speedup vs baseline: 1.0039x; 1.0039x over previous
import jax
import jax.numpy as jnp
from jax.experimental import pallas as pl


def _body(x_ref, kv_ref, m_ref, o_ref):
    wt = kv_ref[...] * m_ref[...]
    o_ref[...] = jnp.dot(wt, x_ref[...], preferred_element_type=jnp.float32)


def kernel(inputs, kernel_values, mask):
    b, c, h, w = inputs.shape
    f = kernel_values.shape[0]
    hw = h * w
    flat_inputs = jnp.reshape(inputs, (c, b * hw))
    out2d = pl.pallas_call(
        _body,
        grid=(b,),
        in_specs=[
            pl.BlockSpec((c, hw), lambda i: (0, i)),
            pl.BlockSpec((f, c), lambda i: (0, 0)),
            pl.BlockSpec((f, c), lambda i: (0, 0)),
        ],
        out_specs=pl.BlockSpec((f, hw), lambda i: (i, 0)),
        out_shape=jax.ShapeDtypeStruct((b * f, hw), jnp.float32),
    )(flat_inputs, kernel_values, mask)
    parts = []
    for i in range(b):
        piece = jax.lax.optimization_barrier(out2d[i * f:(i + 1) * f])
        parts.append(jnp.reshape(piece, (f, h, w)))
    return jnp.stack(parts, axis=0)


# speedup vs baseline: 1.0462x; 1.0422x over previous
import jax
import jax.numpy as jnp
from jax.experimental import pallas as pl


def _body(x_ref, kv_ref, m_ref, o_ref):
    wt = (kv_ref[...] * m_ref[...]).astype(jnp.bfloat16)
    o_ref[...] = jnp.dot(wt, x_ref[...], preferred_element_type=jnp.float32)


def kernel(inputs, kernel_values, mask):
    b, c, h, w = inputs.shape
    f = kernel_values.shape[0]
    hw = h * w
    flat_inputs = jnp.reshape(inputs.astype(jnp.bfloat16), (c, b * hw))
    out2d = pl.pallas_call(
        _body,
        grid=(b,),
        in_specs=[
            pl.BlockSpec((c, hw), lambda i: (0, i)),
            pl.BlockSpec((f, c), lambda i: (0, 0)),
            pl.BlockSpec((f, c), lambda i: (0, 0)),
        ],
        out_specs=pl.BlockSpec((f, hw), lambda i: (i, 0)),
        out_shape=jax.ShapeDtypeStruct((b * f, hw), jnp.float32),
    )(flat_inputs, kernel_values, mask)
    return jnp.reshape(out2d, (b, f, h, w))


# speedup vs baseline: 1.6563x; 1.5831x over previous
import jax
import jax.numpy as jnp
from jax.experimental import pallas as pl


def _body(x_ref, kv_ref, m_ref, o_ref):
    wt = (kv_ref[...] * m_ref[...]).astype(jnp.bfloat16)
    x = x_ref[0]
    o_ref[0] = jax.lax.dot_general(
        x, wt,
        dimension_numbers=(((1,), (1,)), ((), ())),
        preferred_element_type=jnp.float32,
    )


def kernel(inputs, kernel_values, mask):
    b, c, h, w = inputs.shape
    f = kernel_values.shape[0]
    hw = h * w
    flat = jnp.reshape(inputs, (c, b * hw))
    xt = jnp.transpose(jnp.reshape(flat, (c, b, hw)), (1, 2, 0))
    xt = xt.astype(jnp.bfloat16)

    z = pl.pallas_call(
        _body,
        grid=(b,),
        in_specs=[
            pl.BlockSpec((1, hw, c), lambda i: (i, 0, 0)),
            pl.BlockSpec((f, c), lambda i: (0, 0)),
            pl.BlockSpec((f, c), lambda i: (0, 0)),
        ],
        out_specs=pl.BlockSpec((1, hw, f), lambda i: (i, 0, 0)),
        out_shape=jax.ShapeDtypeStruct((b, hw, f), jnp.float32),
    )(xt, kernel_values, mask)

    out = jnp.transpose(z, (0, 2, 1))
    return jnp.reshape(out, (b, f, h, w))
